# baseline (speedup 1.0000x reference)
import jax
import jax.numpy as jnp
from jax import lax
from jax.experimental import pallas as pl
from jax.experimental.pallas import tpu as pltpu
import jax.experimental.pallas.tpu_sc as plsc

_N = 10000
_C = 128
_NC = 2
_NS = 16
_NW = _NC * _NS
_B = 64
_G = 16
_NCH = 10
_NBUF = 4
_NCH_A = 4
_NCH_B = 2 * _NCH - _NCH_A
_TCH = _NS * (_NCH_A + _NCH_B)
_EB = _G * _NCH
_EPP = _B * _EB
_EPAD = _NW * _EPP
_NPAD = 10240
_RPT = _NPAD // _NS

_mesh = plsc.VectorSubcoreMesh(core_axis_name="c", subcore_axis_name="s",
                               num_cores=_NC, num_subcores=_NS)


def _deg_body(dst_hbm, out_hbm, dst_v, ones_v, stage_v, deg_sh):
    c = lax.axis_index("c")
    s = lax.axis_index("s")
    w = s * _NC + c

    def _fill_zero(i, _):
        stage_v[pl.ds(i * 16, 16)] = jnp.zeros((16,), jnp.float32)
        return 0
    lax.fori_loop(0, _RPT // 16, _fill_zero, 0)

    def _fill_one(i, _):
        ones_v[pl.ds(i * 16, 16)] = jnp.ones((16,), jnp.float32)
        return 0
    lax.fori_loop(0, _B // 16, _fill_one, 0)

    pltpu.sync_copy(stage_v, deg_sh.at[pl.ds(s * _RPT, _RPT)])
    plsc.subcore_barrier()

    pltpu.sync_copy(dst_hbm.at[w], dst_v)

    def _scatter(b, _):
        pltpu.sync_copy(ones_v, deg_sh.at[dst_v.at[b]], add=True)
        return 0
    lax.fori_loop(0, _EB, _scatter, 0)
    plsc.subcore_barrier()

    pltpu.sync_copy(deg_sh.at[pl.ds(s * _RPT, _RPT)], stage_v)
    pltpu.sync_copy(stage_v, out_hbm.at[c, pl.ds(s * _RPT, _RPT)])


def _deg_call(dstp):
    return pl.kernel(
        _deg_body,
        out_type=jax.ShapeDtypeStruct((_NC, _NPAD), jnp.float32),
        mesh=_mesh,
        scratch_types=[
            pltpu.VMEM((_EB, _B), jnp.int32),
            pltpu.VMEM((_B,), jnp.float32),
            pltpu.VMEM((_RPT,), jnp.float32),
            pltpu.VMEM_SHARED((_NPAD,), jnp.float32),
        ],
    )(dstp)


def _agg_body(ix_hbm, y_hbm, out_hbm,
              ix_a, ix_b, bufs, acc_sh, sem_i, *sems):
    c = lax.axis_index("c")
    s = lax.axis_index("s")
    w = s * _NC + c
    buf0 = bufs[0]

    def _zero(i, _):
        buf0[i // 8, pl.ds((i % 8) * 16, 16)] = jnp.zeros((16,), jnp.float32)
        return 0
    lax.fori_loop(0, _B * (_C // 16), _zero, 0)

    for k in range(_RPT // _B):
        pltpu.sync_copy(buf0, acc_sh.at[pl.ds(s * _RPT + k * _B, _B)])
    plsc.subcore_barrier()

    def _chunk(ix_cur):
        descs = {}
        for t in range(min(_NBUF - 1, _G)):
            descs[t] = pltpu.async_copy(y_hbm.at[ix_cur.at[0, t]],
                                        bufs[t % _NBUF], sems[t % _NBUF])
        for t in range(_G):
            tp = t + _NBUF - 1
            if tp < _G:
                descs[tp] = pltpu.async_copy(y_hbm.at[ix_cur.at[0, tp]],
                                             bufs[tp % _NBUF], sems[tp % _NBUF])
            descs.pop(t).wait()
            pltpu.sync_copy(bufs[t % _NBUF], acc_sh.at[ix_cur.at[1, t]],
                            add=True)

    n_half = jnp.where(c == 0, _NCH_A // 2, _NCH_B // 2)
    base = jnp.where(c == 0, s * _NCH_A, _NS * _NCH_A + s * _NCH_B)
    last = base + 2 * n_half - 1

    pltpu.sync_copy(ix_hbm.at[base], ix_a)

    def _body(i, _):
        i2 = 2 * i
        d1 = pltpu.async_copy(ix_hbm.at[base + i2 + 1], ix_b, sem_i)
        _chunk(ix_a)
        d1.wait()
        d2 = pltpu.async_copy(ix_hbm.at[jnp.minimum(base + i2 + 2, last)],
                              ix_a, sem_i)
        _chunk(ix_b)
        d2.wait()
        return 0
    lax.fori_loop(0, n_half, _body, 0)
    plsc.subcore_barrier()

    for k in range(_RPT // _B):
        base = s * _RPT + k * _B
        pltpu.sync_copy(acc_sh.at[pl.ds(base, _B)], buf0)
        pltpu.sync_copy(buf0, out_hbm.at[c, pl.ds(base, _B)])


def _agg_call(ixp, y):
    return pl.kernel(
        _agg_body,
        out_type=jax.ShapeDtypeStruct((_NC, _NPAD, _C), jnp.float32),
        mesh=_mesh,
        scratch_types=[
            pltpu.VMEM((2, _G, _B), jnp.int32),
            pltpu.VMEM((2, _G, _B), jnp.int32),
            [pltpu.VMEM((_B, _C), jnp.float32)] * _NBUF,
            pltpu.VMEM_SHARED((_NPAD, _C), jnp.float32),
            pltpu.SemaphoreType.DMA,
        ] + [pltpu.SemaphoreType.DMA] * _NBUF,
    )(ixp, y)


def _pre_body(x_ref, degp_ref, w0_ref, wih_ref, whh_ref, bih_ref, bhh_ref,
              y_ref):
    w0 = w0_ref[...]
    gi = lax.dot_general(w0, wih_ref[...], (((1,), (1,)), ((), ()))) + bih_ref[...][None, :]
    gh = lax.dot_general(w0, whh_ref[...], (((1,), (1,)), ((), ()))) + bhh_ref[...][None, :]
    r = jax.nn.sigmoid(gi[:, :_C] + gh[:, :_C])
    z = jax.nn.sigmoid(gi[:, _C:2 * _C] + gh[:, _C:2 * _C])
    n = jnp.tanh(gi[:, 2 * _C:] + r * gh[:, 2 * _C:])
    wev = (1.0 - z) * n + z * w0
    xw = jnp.dot(x_ref[...], wev, preferred_element_type=jnp.float32)
    deg = 1.0 + degp_ref[0, :_N] + degp_ref[1, :_N]
    dinv = lax.rsqrt(deg)
    y_ref[...] = xw * dinv[:, None]


def _pre_call(x, degp, w0, wih, whh, bih, bhh):
    return pl.pallas_call(
        _pre_body,
        out_shape=jax.ShapeDtypeStruct((_N, _C), jnp.float32),
    )(x, degp, w0, wih, whh, bih, bhh)


def _post_body(accp_ref, y_ref, degp_ref, wcls_ref, bcls_ref, out_ref):
    deg = 1.0 + degp_ref[0, :_N] + degp_ref[1, :_N]
    dinv = lax.rsqrt(deg)
    total = accp_ref[0, :_N, :] + accp_ref[1, :_N, :] + y_ref[...]
    h = jnp.maximum(total * dinv[:, None], 0.0)
    out = lax.dot_general(h, wcls_ref[...], (((1,), (1,)), ((), ())))
    out_ref[...] = out + bcls_ref[...][None, :]


def _post_call(accp, y, degp, wcls, bcls):
    return pl.pallas_call(
        _post_body,
        out_shape=jax.ShapeDtypeStruct((_N, wcls.shape[0]), jnp.float32),
    )(accp, y, degp, wcls, bcls)


def kernel(x, edge_index, initial_weight, W_ih, W_hh, b_ih, b_hh, W_cls, b_cls):
    src = edge_index[0]
    dst = edge_index[1]
    pad = _EPAD - src.shape[0]
    srcp = jnp.concatenate([src, jnp.zeros((pad,), jnp.int32)])
    dstp = jnp.concatenate([dst, jnp.full((pad,), _N, jnp.int32)])
    ixp = jnp.stack([srcp.reshape(_TCH, _G, _B),
                     dstp.reshape(_TCH, _G, _B)], axis=1)
    degp = _deg_call(dstp.reshape(_NW, _EB, _B))
    y = _pre_call(x, degp, initial_weight, W_ih, W_hh, b_ih, b_hh)
    accp = _agg_call(ixp, y)
    return _post_call(accp, y, degp, W_cls, b_cls)


# speedup vs baseline: 1.1238x; 1.1238x over previous
import jax
import jax.numpy as jnp
from jax import lax
from jax.experimental import pallas as pl
from jax.experimental.pallas import tpu as pltpu
import jax.experimental.pallas.tpu_sc as plsc

_N = 10000
_C = 128
_NC = 2
_NS = 16
_NW = _NC * _NS
_B = 64
_G = 16
_NCH = 10
_NBUF = 4
_NCH_A = 14
_NCH_B = 2 * _NCH - _NCH_A
_TCH = _NS * (_NCH_A + _NCH_B)
_EB = _G * _NCH
_EPP = _B * _EB
_EPAD = _NW * _EPP
_NPAD = 10240
_RPT = _NPAD // _NS

_mesh = plsc.VectorSubcoreMesh(core_axis_name="c", subcore_axis_name="s",
                               num_cores=_NC, num_subcores=_NS)


def _deg_body(dst_hbm, out_hbm, dst_v, ones_v, stage_v, deg_sh):
    c = lax.axis_index("c")
    s = lax.axis_index("s")
    w = s * _NC + c

    def _fill_zero(i, _):
        stage_v[pl.ds(i * 16, 16)] = jnp.zeros((16,), jnp.float32)
        return 0
    lax.fori_loop(0, _RPT // 16, _fill_zero, 0)

    def _fill_one(i, _):
        ones_v[pl.ds(i * 16, 16)] = jnp.ones((16,), jnp.float32)
        return 0
    lax.fori_loop(0, _B // 16, _fill_one, 0)

    pltpu.sync_copy(stage_v, deg_sh.at[pl.ds(s * _RPT, _RPT)])
    plsc.subcore_barrier()

    pltpu.sync_copy(dst_hbm.at[w], dst_v)

    def _scatter(b, _):
        pltpu.sync_copy(ones_v, deg_sh.at[dst_v.at[b]], add=True)
        return 0
    lax.fori_loop(0, _EB, _scatter, 0)
    plsc.subcore_barrier()

    pltpu.sync_copy(deg_sh.at[pl.ds(s * _RPT, _RPT)], stage_v)
    pltpu.sync_copy(stage_v, out_hbm.at[c, pl.ds(s * _RPT, _RPT)])


def _deg_call(dstp):
    return pl.kernel(
        _deg_body,
        out_type=jax.ShapeDtypeStruct((_NC, _NPAD), jnp.float32),
        mesh=_mesh,
        scratch_types=[
            pltpu.VMEM((_EB, _B), jnp.int32),
            pltpu.VMEM((_B,), jnp.float32),
            pltpu.VMEM((_RPT,), jnp.float32),
            pltpu.VMEM_SHARED((_NPAD,), jnp.float32),
        ],
    )(dstp)


def _agg_body(ix_hbm, y_hbm, out_hbm,
              ix_a, ix_b, bufs, acc_sh, sem_i, *sems):
    c = lax.axis_index("c")
    s = lax.axis_index("s")
    w = s * _NC + c
    buf0 = bufs[0]

    def _zero(i, _):
        buf0[i // 8, pl.ds((i % 8) * 16, 16)] = jnp.zeros((16,), jnp.float32)
        return 0
    lax.fori_loop(0, _B * (_C // 16), _zero, 0)

    for k in range(_RPT // _B):
        pltpu.sync_copy(buf0, acc_sh.at[pl.ds(s * _RPT + k * _B, _B)])
    plsc.subcore_barrier()

    def _chunk(ix_cur):
        descs = {}
        for t in range(min(_NBUF - 1, _G)):
            descs[t] = pltpu.async_copy(y_hbm.at[ix_cur.at[0, t]],
                                        bufs[t % _NBUF], sems[t % _NBUF])
        for t in range(_G):
            tp = t + _NBUF - 1
            if tp < _G:
                descs[tp] = pltpu.async_copy(y_hbm.at[ix_cur.at[0, tp]],
                                             bufs[tp % _NBUF], sems[tp % _NBUF])
            descs.pop(t).wait()
            pltpu.sync_copy(bufs[t % _NBUF], acc_sh.at[ix_cur.at[1, t]],
                            add=True)

    n_half = jnp.where(c == 0, _NCH_A // 2, _NCH_B // 2)
    base = jnp.where(c == 0, s * _NCH_A, _NS * _NCH_A + s * _NCH_B)
    last = base + 2 * n_half - 1

    pltpu.sync_copy(ix_hbm.at[base], ix_a)

    def _body(i, _):
        i2 = 2 * i
        d1 = pltpu.async_copy(ix_hbm.at[base + i2 + 1], ix_b, sem_i)
        _chunk(ix_a)
        d1.wait()
        d2 = pltpu.async_copy(ix_hbm.at[jnp.minimum(base + i2 + 2, last)],
                              ix_a, sem_i)
        _chunk(ix_b)
        d2.wait()
        return 0
    lax.fori_loop(0, n_half, _body, 0)
    plsc.subcore_barrier()

    for k in range(_RPT // _B):
        base = s * _RPT + k * _B
        pltpu.sync_copy(acc_sh.at[pl.ds(base, _B)], buf0)
        pltpu.sync_copy(buf0, out_hbm.at[c, pl.ds(base, _B)])


def _agg_call(ixp, y):
    return pl.kernel(
        _agg_body,
        out_type=jax.ShapeDtypeStruct((_NC, _NPAD, _C), jnp.float32),
        mesh=_mesh,
        scratch_types=[
            pltpu.VMEM((2, _G, _B), jnp.int32),
            pltpu.VMEM((2, _G, _B), jnp.int32),
            [pltpu.VMEM((_B, _C), jnp.float32)] * _NBUF,
            pltpu.VMEM_SHARED((_NPAD, _C), jnp.float32),
            pltpu.SemaphoreType.DMA,
        ] + [pltpu.SemaphoreType.DMA] * _NBUF,
    )(ixp, y)


def _pre_body(x_ref, degp_ref, w0_ref, wih_ref, whh_ref, bih_ref, bhh_ref,
              y_ref):
    w0 = w0_ref[...]
    gi = lax.dot_general(w0, wih_ref[...], (((1,), (1,)), ((), ()))) + bih_ref[...][None, :]
    gh = lax.dot_general(w0, whh_ref[...], (((1,), (1,)), ((), ()))) + bhh_ref[...][None, :]
    r = jax.nn.sigmoid(gi[:, :_C] + gh[:, :_C])
    z = jax.nn.sigmoid(gi[:, _C:2 * _C] + gh[:, _C:2 * _C])
    n = jnp.tanh(gi[:, 2 * _C:] + r * gh[:, 2 * _C:])
    wev = (1.0 - z) * n + z * w0
    xw = jnp.dot(x_ref[...], wev, preferred_element_type=jnp.float32)
    deg = 1.0 + degp_ref[0, :_N] + degp_ref[1, :_N]
    dinv = lax.rsqrt(deg)
    y_ref[...] = xw * dinv[:, None]


def _pre_call(x, degp, w0, wih, whh, bih, bhh):
    return pl.pallas_call(
        _pre_body,
        out_shape=jax.ShapeDtypeStruct((_N, _C), jnp.float32),
    )(x, degp, w0, wih, whh, bih, bhh)


def _post_body(accp_ref, y_ref, degp_ref, wcls_ref, bcls_ref, out_ref):
    deg = 1.0 + degp_ref[0, :_N] + degp_ref[1, :_N]
    dinv = lax.rsqrt(deg)
    total = accp_ref[0, :_N, :] + accp_ref[1, :_N, :] + y_ref[...]
    h = jnp.maximum(total * dinv[:, None], 0.0)
    out = lax.dot_general(h, wcls_ref[...], (((1,), (1,)), ((), ())))
    out_ref[...] = out + bcls_ref[...][None, :]


def _post_call(accp, y, degp, wcls, bcls):
    return pl.pallas_call(
        _post_body,
        out_shape=jax.ShapeDtypeStruct((_N, wcls.shape[0]), jnp.float32),
    )(accp, y, degp, wcls, bcls)


def kernel(x, edge_index, initial_weight, W_ih, W_hh, b_ih, b_hh, W_cls, b_cls):
    src = edge_index[0]
    dst = edge_index[1]
    pad = _EPAD - src.shape[0]
    srcp = jnp.concatenate([src, jnp.zeros((pad,), jnp.int32)])
    dstp = jnp.concatenate([dst, jnp.full((pad,), _N, jnp.int32)])
    ixp = jnp.stack([srcp.reshape(_TCH, _G, _B),
                     dstp.reshape(_TCH, _G, _B)], axis=1)
    degp = _deg_call(dstp.reshape(_NW, _EB, _B))
    y = _pre_call(x, degp, initial_weight, W_ih, W_hh, b_ih, b_hh)
    accp = _agg_call(ixp, y)
    return _post_call(accp, y, degp, W_cls, b_cls)


# speedup vs baseline: 1.3060x; 1.1622x over previous
import jax
import jax.numpy as jnp
from jax import lax
from jax.experimental import pallas as pl
from jax.experimental.pallas import tpu as pltpu
import jax.experimental.pallas.tpu_sc as plsc

_N = 10000
_C = 128
_NC = 2
_NS = 16
_NW = _NC * _NS
_B = 64
_G = 16
_NCH = 10
_NBUF = 4
_NCH_A = 18
_NCH_B = 2 * _NCH - _NCH_A
_TCH = _NS * (_NCH_A + _NCH_B)
_EB = _G * _NCH
_EPP = _B * _EB
_EPAD = _NW * _EPP
_NPAD = 10240
_RPT = _NPAD // _NS

_mesh = plsc.VectorSubcoreMesh(core_axis_name="c", subcore_axis_name="s",
                               num_cores=_NC, num_subcores=_NS)


def _deg_body(dst_hbm, out_hbm, dst_v, ones_v, stage_v, deg_sh):
    c = lax.axis_index("c")
    s = lax.axis_index("s")
    w = s * _NC + c

    def _fill_zero(i, _):
        stage_v[pl.ds(i * 16, 16)] = jnp.zeros((16,), jnp.float32)
        return 0
    lax.fori_loop(0, _RPT // 16, _fill_zero, 0)

    def _fill_one(i, _):
        ones_v[pl.ds(i * 16, 16)] = jnp.ones((16,), jnp.float32)
        return 0
    lax.fori_loop(0, _B // 16, _fill_one, 0)

    pltpu.sync_copy(stage_v, deg_sh.at[pl.ds(s * _RPT, _RPT)])
    plsc.subcore_barrier()

    pltpu.sync_copy(dst_hbm.at[w], dst_v)

    def _scatter(b, _):
        pltpu.sync_copy(ones_v, deg_sh.at[dst_v.at[b]], add=True)
        return 0
    lax.fori_loop(0, _EB, _scatter, 0)
    plsc.subcore_barrier()

    pltpu.sync_copy(deg_sh.at[pl.ds(s * _RPT, _RPT)], stage_v)
    pltpu.sync_copy(stage_v, out_hbm.at[c, pl.ds(s * _RPT, _RPT)])


def _deg_call(dstp):
    return pl.kernel(
        _deg_body,
        out_type=jax.ShapeDtypeStruct((_NC, _NPAD), jnp.float32),
        mesh=_mesh,
        scratch_types=[
            pltpu.VMEM((_EB, _B), jnp.int32),
            pltpu.VMEM((_B,), jnp.float32),
            pltpu.VMEM((_RPT,), jnp.float32),
            pltpu.VMEM_SHARED((_NPAD,), jnp.float32),
        ],
    )(dstp)


def _agg_body(ix_hbm, y_hbm, out_hbm,
              ix_a, ix_b, bufs, acc_sh, sem_i, *sems):
    c = lax.axis_index("c")
    s = lax.axis_index("s")
    w = s * _NC + c
    buf0 = bufs[0]

    def _zero(i, _):
        buf0[i // 8, pl.ds((i % 8) * 16, 16)] = jnp.zeros((16,), jnp.float32)
        return 0
    lax.fori_loop(0, _B * (_C // 16), _zero, 0)

    for k in range(_RPT // _B):
        pltpu.sync_copy(buf0, acc_sh.at[pl.ds(s * _RPT + k * _B, _B)])
    plsc.subcore_barrier()

    def _chunk(ix_cur):
        descs = {}
        for t in range(min(_NBUF - 1, _G)):
            descs[t] = pltpu.async_copy(y_hbm.at[ix_cur.at[0, t]],
                                        bufs[t % _NBUF], sems[t % _NBUF])
        for t in range(_G):
            tp = t + _NBUF - 1
            if tp < _G:
                descs[tp] = pltpu.async_copy(y_hbm.at[ix_cur.at[0, tp]],
                                             bufs[tp % _NBUF], sems[tp % _NBUF])
            descs.pop(t).wait()
            pltpu.sync_copy(bufs[t % _NBUF], acc_sh.at[ix_cur.at[1, t]],
                            add=True)

    n_half = jnp.where(c == 0, _NCH_A // 2, _NCH_B // 2)
    base = jnp.where(c == 0, s * _NCH_A, _NS * _NCH_A + s * _NCH_B)
    last = base + 2 * n_half - 1

    pltpu.sync_copy(ix_hbm.at[base], ix_a)

    def _body(i, _):
        i2 = 2 * i
        d1 = pltpu.async_copy(ix_hbm.at[base + i2 + 1], ix_b, sem_i)
        _chunk(ix_a)
        d1.wait()
        d2 = pltpu.async_copy(ix_hbm.at[jnp.minimum(base + i2 + 2, last)],
                              ix_a, sem_i)
        _chunk(ix_b)
        d2.wait()
        return 0
    lax.fori_loop(0, n_half, _body, 0)
    plsc.subcore_barrier()

    for k in range(_RPT // _B):
        base = s * _RPT + k * _B
        pltpu.sync_copy(acc_sh.at[pl.ds(base, _B)], buf0)
        pltpu.sync_copy(buf0, out_hbm.at[c, pl.ds(base, _B)])


def _agg_call(ixp, y):
    return pl.kernel(
        _agg_body,
        out_type=jax.ShapeDtypeStruct((_NC, _NPAD, _C), jnp.float32),
        mesh=_mesh,
        scratch_types=[
            pltpu.VMEM((2, _G, _B), jnp.int32),
            pltpu.VMEM((2, _G, _B), jnp.int32),
            [pltpu.VMEM((_B, _C), jnp.float32)] * _NBUF,
            pltpu.VMEM_SHARED((_NPAD, _C), jnp.float32),
            pltpu.SemaphoreType.DMA,
        ] + [pltpu.SemaphoreType.DMA] * _NBUF,
    )(ixp, y)


def _pre_body(x_ref, degp_ref, w0_ref, wih_ref, whh_ref, bih_ref, bhh_ref,
              y_ref):
    w0 = w0_ref[...]
    gi = lax.dot_general(w0, wih_ref[...], (((1,), (1,)), ((), ()))) + bih_ref[...][None, :]
    gh = lax.dot_general(w0, whh_ref[...], (((1,), (1,)), ((), ()))) + bhh_ref[...][None, :]
    r = jax.nn.sigmoid(gi[:, :_C] + gh[:, :_C])
    z = jax.nn.sigmoid(gi[:, _C:2 * _C] + gh[:, _C:2 * _C])
    n = jnp.tanh(gi[:, 2 * _C:] + r * gh[:, 2 * _C:])
    wev = (1.0 - z) * n + z * w0
    xw = jnp.dot(x_ref[...], wev, preferred_element_type=jnp.float32)
    deg = 1.0 + degp_ref[0, :_N] + degp_ref[1, :_N]
    dinv = lax.rsqrt(deg)
    y_ref[...] = xw * dinv[:, None]


def _pre_call(x, degp, w0, wih, whh, bih, bhh):
    return pl.pallas_call(
        _pre_body,
        out_shape=jax.ShapeDtypeStruct((_N, _C), jnp.float32),
    )(x, degp, w0, wih, whh, bih, bhh)


def _post_body(accp_ref, y_ref, degp_ref, wcls_ref, bcls_ref, out_ref):
    deg = 1.0 + degp_ref[0, :_N] + degp_ref[1, :_N]
    dinv = lax.rsqrt(deg)
    total = accp_ref[0, :_N, :] + accp_ref[1, :_N, :] + y_ref[...]
    h = jnp.maximum(total * dinv[:, None], 0.0)
    out = lax.dot_general(h, wcls_ref[...], (((1,), (1,)), ((), ())))
    out_ref[...] = out + bcls_ref[...][None, :]


def _post_call(accp, y, degp, wcls, bcls):
    return pl.pallas_call(
        _post_body,
        out_shape=jax.ShapeDtypeStruct((_N, wcls.shape[0]), jnp.float32),
    )(accp, y, degp, wcls, bcls)


def kernel(x, edge_index, initial_weight, W_ih, W_hh, b_ih, b_hh, W_cls, b_cls):
    src = edge_index[0]
    dst = edge_index[1]
    pad = _EPAD - src.shape[0]
    srcp = jnp.concatenate([src, jnp.zeros((pad,), jnp.int32)])
    dstp = jnp.concatenate([dst, jnp.full((pad,), _N, jnp.int32)])
    ixp = jnp.stack([srcp.reshape(_TCH, _G, _B),
                     dstp.reshape(_TCH, _G, _B)], axis=1)
    degp = _deg_call(dstp.reshape(_NW, _EB, _B))
    y = _pre_call(x, degp, initial_weight, W_ih, W_hh, b_ih, b_hh)
    accp = _agg_call(ixp, y)
    return _post_call(accp, y, degp, W_cls, b_cls)
